# baseline (speedup 1.0000x reference)
import numpy as np
import jax
import jax.numpy as jnp
from jax.experimental import pallas as pl


def _masked_idx(c: int, ratio: float) -> list:
    mask = np.linspace(0, c * (1 - ratio), int(c * ratio))
    return [i for i in range(c) if i in mask]


def _copy_mask_kernel(x_ref, n_ref, o_ref):
    x = x_ref[0]
    nz = n_ref[0, 0]
    row = jax.lax.broadcasted_iota(jnp.int32, x.shape, 0)
    o_ref[0] = jnp.where(row == 0, nz, x)


def kernel(input1, noise):
    b, c, h, w = input1.shape
    hw = h * w
    idx = _masked_idx(c, 0.5)
    nmask = len(idx)
    cb = c // nmask
    if idx != [j * cb for j in range(nmask)]:
        raise ValueError("masked channels not uniformly spaced")

    x = input1.reshape(b, c, hw)
    nz = noise.reshape(nmask, b, 1, hw)
    out = pl.pallas_call(
        _copy_mask_kernel,
        grid=(b, nmask),
        in_specs=[
            pl.BlockSpec((1, cb, hw), lambda i, j: (i, j, 0)),
            pl.BlockSpec((1, 1, 1, hw), lambda i, j: (j, i, 0, 0)),
        ],
        out_specs=pl.BlockSpec((1, cb, hw), lambda i, j: (i, j, 0)),
        out_shape=jax.ShapeDtypeStruct((b, c, hw), x.dtype),
    )(x, nz)
    return out.reshape(b, c, h, w)
